# baseline (speedup 1.0000x reference)
import jax
import jax.numpy as jnp
from jax import lax
from jax.experimental import pallas as pl
from jax.experimental.pallas import tpu as pltpu
from jax.experimental.pallas import tpu_sc as plsc

_B, _S, _D = 4, 4096, 2048
_NC, _NS, _L = 2, 16, 16
_NW = _NC * _NS
_ROWS_PER_W = _S // _NW
_CHUNK = 4
_N_CHUNKS = _ROWS_PER_W // _CHUNK


def _sc_body(x_hbm, pos_hbm, out_hbm,
             xb0, xb1, ob0, ob1, pb0, pb1,
             sx0, sx1, so0, so1, sp0, sp1):
    wid = lax.axis_index("s") * _NC + lax.axis_index("c")
    base = wid * _ROWS_PER_W
    xb, ob, pb = (xb0, xb1), (ob0, ob1), (pb0, pb1)
    sx, so, sp = (sx0, sx1), (so0, so1), (sp0, sp1)

    def x_src(ci, b):
        return x_hbm.at[b, pl.ds(base + ci * _CHUNK, _CHUNK)]

    def o_dst(ci, b):
        return out_hbm.at[b, pl.ds(base + ci * _CHUNK, _CHUNK)]

    def p_src(ci):
        return pos_hbm.at[pl.ds(base + ci * _CHUNK, _CHUNK)]

    def compute(p, q):
        for r in range(_CHUNK):
            @plsc.parallel_loop(0, _D, _L, unroll=8)
            def _(i):
                sl = pl.ds(i, _L)
                ob[p][r, sl] = xb[p][r, sl] + pb[q][r, sl]

    def step(ci, b, q, skip_store_wait=False, next_ci_ok=True):
        p = b % 2
        pltpu.make_async_copy(x_src(ci, b), xb[p], sx[p]).wait()
        if b == 0:
            pltpu.make_async_copy(p_src(ci), pb[q], sp[q]).wait()
        if not skip_store_wait:
            pltpu.make_async_copy(ob[p], o_dst(ci, b), so[p]).wait()
        compute(p, q)
        pltpu.async_copy(ob[p], o_dst(ci, b), so[p])
        if b < 2:
            pltpu.async_copy(x_src(ci, b + 2), xb[p], sx[p])
        else:
            def _fire_next():
                pltpu.async_copy(x_src(ci + 1, b - 2), xb[p], sx[p])
            if next_ci_ok is True:
                _fire_next()
            else:
                pl.when(next_ci_ok)(_fire_next)

    def block(ci, q, first=False, next_ci_ok=True):
        def _fire_pos():
            pltpu.async_copy(p_src(ci + 1), pb[1 - q], sp[1 - q])
        if next_ci_ok is True:
            _fire_pos()
        else:
            pl.when(next_ci_ok)(_fire_pos)
        step(ci, 0, q, skip_store_wait=first)
        step(ci, 1, q, skip_store_wait=first, next_ci_ok=True)
        step(ci, 2, q, next_ci_ok=next_ci_ok)
        step(ci, 3, q, next_ci_ok=next_ci_ok)

    pltpu.async_copy(p_src(0), pb[0], sp[0])
    pltpu.async_copy(x_src(0, 0), xb[0], sx[0])
    pltpu.async_copy(x_src(0, 1), xb[1], sx[1])

    block(0, 0, first=True)
    block(1, 1)

    def k_body(k, carry):
        ci = 2 * k
        block(ci, 0)
        block(ci + 1, 1, next_ci_ok=(k < _N_CHUNKS // 2 - 1))
        return carry

    lax.fori_loop(1, _N_CHUNKS // 2, k_body, 0)

    last = _N_CHUNKS - 1
    pltpu.make_async_copy(ob[0], o_dst(last, 2), so[0]).wait()
    pltpu.make_async_copy(ob[1], o_dst(last, 3), so[1]).wait()


def kernel(x, pos_table):
    mesh = plsc.VectorSubcoreMesh(
        core_axis_name="c", subcore_axis_name="s",
        num_cores=_NC, num_subcores=_NS,
    )
    buf = pltpu.VMEM((_CHUNK, _D), jnp.float32)
    sem = pltpu.SemaphoreType.DMA
    return pl.kernel(
        _sc_body,
        out_type=jax.ShapeDtypeStruct((_B, _S, _D), jnp.float32),
        mesh=mesh,
        scratch_types=[buf, buf, buf, buf, buf, buf,
                       sem, sem, sem, sem, sem, sem],
    )(x, pos_table)


# speedup vs baseline: 1.0931x; 1.0931x over previous
import jax
import jax.numpy as jnp
from jax import lax
from jax.experimental import pallas as pl
from jax.experimental.pallas import tpu as pltpu
from jax.experimental.pallas import tpu_sc as plsc

_B, _S, _D = 4, 4096, 2048
_NC, _NS, _L = 2, 16, 16
_NW = _NC * _NS
_ROWS_PER_W = _S // _NW
_CHUNK = 8
_N_CHUNKS = _ROWS_PER_W // _CHUNK


def _sc_body(x_hbm, pos_hbm, out_hbm,
             xb0, xb1, ob0, ob1, pb0, pb1,
             sx0, sx1, so0, so1, sp0, sp1):
    wid = lax.axis_index("s") * _NC + lax.axis_index("c")
    base = wid * _ROWS_PER_W
    xb, ob, pb = (xb0, xb1), (ob0, ob1), (pb0, pb1)
    sx, so, sp = (sx0, sx1), (so0, so1), (sp0, sp1)

    def x_src(ci, b):
        return x_hbm.at[b, pl.ds(base + ci * _CHUNK, _CHUNK)]

    def o_dst(ci, b):
        return out_hbm.at[b, pl.ds(base + ci * _CHUNK, _CHUNK)]

    def p_src(ci):
        return pos_hbm.at[pl.ds(base + ci * _CHUNK, _CHUNK)]

    def compute(p, q):
        for r in range(_CHUNK):
            @plsc.parallel_loop(0, _D, _L, unroll=8)
            def _(i):
                sl = pl.ds(i, _L)
                ob[p][r, sl] = xb[p][r, sl] + pb[q][r, sl]

    def step(ci, b, q, skip_store_wait=False, next_ci_ok=True):
        p = b % 2
        pltpu.make_async_copy(x_src(ci, b), xb[p], sx[p]).wait()
        if b == 0:
            pltpu.make_async_copy(p_src(ci), pb[q], sp[q]).wait()
        if not skip_store_wait:
            pltpu.make_async_copy(ob[p], o_dst(ci, b), so[p]).wait()
        compute(p, q)
        pltpu.async_copy(ob[p], o_dst(ci, b), so[p])
        if b < 2:
            pltpu.async_copy(x_src(ci, b + 2), xb[p], sx[p])
        else:
            def _fire_next():
                pltpu.async_copy(x_src(ci + 1, b - 2), xb[p], sx[p])
            if next_ci_ok is True:
                _fire_next()
            else:
                pl.when(next_ci_ok)(_fire_next)

    def block(ci, q, first=False, next_ci_ok=True):
        def _fire_pos():
            pltpu.async_copy(p_src(ci + 1), pb[1 - q], sp[1 - q])
        if next_ci_ok is True:
            _fire_pos()
        else:
            pl.when(next_ci_ok)(_fire_pos)
        step(ci, 0, q, skip_store_wait=first)
        step(ci, 1, q, skip_store_wait=first, next_ci_ok=True)
        step(ci, 2, q, next_ci_ok=next_ci_ok)
        step(ci, 3, q, next_ci_ok=next_ci_ok)

    pltpu.async_copy(p_src(0), pb[0], sp[0])
    pltpu.async_copy(x_src(0, 0), xb[0], sx[0])
    pltpu.async_copy(x_src(0, 1), xb[1], sx[1])

    block(0, 0, first=True)
    block(1, 1)

    def k_body(k, carry):
        ci = 2 * k
        block(ci, 0)
        block(ci + 1, 1, next_ci_ok=(k < _N_CHUNKS // 2 - 1))
        return carry

    lax.fori_loop(1, _N_CHUNKS // 2, k_body, 0)

    last = _N_CHUNKS - 1
    pltpu.make_async_copy(ob[0], o_dst(last, 2), so[0]).wait()
    pltpu.make_async_copy(ob[1], o_dst(last, 3), so[1]).wait()


def kernel(x, pos_table):
    mesh = plsc.VectorSubcoreMesh(
        core_axis_name="c", subcore_axis_name="s",
        num_cores=_NC, num_subcores=_NS,
    )
    buf = pltpu.VMEM((_CHUNK, _D), jnp.float32)
    sem = pltpu.SemaphoreType.DMA
    return pl.kernel(
        _sc_body,
        out_type=jax.ShapeDtypeStruct((_B, _S, _D), jnp.float32),
        mesh=mesh,
        scratch_types=[buf, buf, buf, buf, buf, buf,
                       sem, sem, sem, sem, sem, sem],
    )(x, pos_table)
